# Initial kernel scaffold; baseline (speedup 1.0000x reference)
import jax
import jax.numpy as jnp
from jax.experimental import pallas as pl


def kernel(x, adjs_edge_index, adjs_edge_weight, W1, b1, W2, b2, A_w, A_b, C_w, C_b):
    raise NotImplementedError("write your pallas kernel here")


# speedup vs baseline: 4.0612x; 4.0612x over previous
import dataclasses
import functools

import jax
import jax.numpy as jnp
from jax import lax
from jax.experimental import pallas as pl
from jax.experimental.pallas import tpu as pltpu
from jax.experimental.pallas import tpu_sc as plsc

_N = 10000
_NFEAT = 128
_NHID = 128
_NCLASS = 2
_NREL = 5
_NADJ = 11
_E = 320000

_LANES = 16
_CHUNK = 128
_CPR = _E // _CHUNK
_CPP = 64
_TOTC = 27520
_PADC = _TOTC - _NADJ * _CPR
_UNITS = _TOTC // _CPP
_NW = 32
_UPW = (_UNITS + _NW - 1) // _NW
_SUB_ROWS = 624
_TAIL_ROWS = _N - 16 * _SUB_ROWS
_NSEG = _NHID // _LANES


def _sc_body(S_hbm, src_hbm, dst_hbm, w_hbm, out_hbm, src_v, dst_v, w_v,
             rows_v, acc):
    cid = lax.axis_index("c")
    sid = lax.axis_index("s")
    wid = sid * 2 + cid
    zeros16 = jnp.zeros((_LANES,), jnp.float32)

    @pl.loop(0, _CHUNK)
    def _(rr):
        for j in range(_NSEG):
            rows_v[rr, pl.ds(j * _LANES, _LANES)] = zeros16

    base = sid * _SUB_ROWS
    for z in range(_SUB_ROWS // _CHUNK):
        pltpu.sync_copy(rows_v, acc.at[pl.ds(base + z * _CHUNK, _CHUNK)])
    rem = _SUB_ROWS % _CHUNK
    if rem:
        pltpu.sync_copy(rows_v.at[pl.ds(0, rem)],
                        acc.at[pl.ds(base + _SUB_ROWS - rem, rem)])

    @pl.when(sid == 0)
    def _():
        pltpu.sync_copy(rows_v.at[pl.ds(0, _TAIL_ROWS)],
                        acc.at[pl.ds(16 * _SUB_ROWS, _TAIL_ROWS)])

    plsc.subcore_barrier()

    @pl.loop(0, _UPW)
    def _(t):
        u = wid + t * _NW

        @pl.when(u < _UNITS)
        def _():
            c0 = u * _CPP
            pltpu.sync_copy(src_hbm.at[pl.ds(c0, _CPP)], src_v)
            pltpu.sync_copy(dst_hbm.at[pl.ds(c0, _CPP)], dst_v)
            pltpu.sync_copy(w_hbm.at[pl.ds(c0, _CPP)], w_v)

            @pl.loop(0, _CPP)
            def _(c):
                pltpu.sync_copy(S_hbm.at[src_v.at[c]], rows_v)
                cidx = jnp.full((_LANES,), c, jnp.int32)

                @pl.loop(0, _CHUNK)
                def _(e):
                    wv = plsc.load_gather(
                        w_v, [cidx, jnp.full((_LANES,), e, jnp.int32)])
                    for j in range(_NSEG):
                        sl = pl.ds(j * _LANES, _LANES)
                        rows_v[e, sl] = rows_v[e, sl] * wv

                pltpu.sync_copy(rows_v, acc.at[dst_v.at[c]], add=True)

    plsc.subcore_barrier()
    pltpu.sync_copy(acc.at[pl.ds(sid * _SUB_ROWS, _SUB_ROWS)],
                    out_hbm.at[pl.ds(cid * _N + sid * _SUB_ROWS, _SUB_ROWS)])

    @pl.when(sid == 0)
    def _():
        pltpu.sync_copy(acc.at[pl.ds(16 * _SUB_ROWS, _TAIL_ROWS)],
                        out_hbm.at[pl.ds(cid * _N + 16 * _SUB_ROWS,
                                         _TAIL_ROWS)])


def _sc_agg(S2d, src2d, dst2d, w2d):
    mesh = plsc.VectorSubcoreMesh(core_axis_name="c", subcore_axis_name="s")
    cp = pltpu.CompilerParams()
    if "needs_layout_passes" in pltpu.CompilerParams.__dataclass_fields__:
        cp = dataclasses.replace(cp, needs_layout_passes=False)
    kern = functools.partial(
        pl.kernel,
        compiler_params=cp,
        out_type=jax.ShapeDtypeStruct((2 * _N, _NHID), jnp.float32),
        mesh=mesh,
        scratch_types=[
            pltpu.VMEM((_CPP, _CHUNK), jnp.int32),
            pltpu.VMEM((_CPP, _CHUNK), jnp.int32),
            pltpu.VMEM((_CPP, _CHUNK), jnp.float32),
            pltpu.VMEM((_CHUNK, _NHID), jnp.float32),
            pltpu.VMEM_SHARED((_N, _NHID), jnp.float32),
        ],
    )(_sc_body)
    return kern(S2d, src2d, dst2d, w2d)


def _mm_rel(P, W, bsum, relu_combine):
    nrel = W.shape[0]
    nrows = P.shape[1]

    def body(p_ref, w_ref, b_ref, out_ref, x1_ref):
        @pl.when(pl.program_id(0) == 0)
        def _():
            if relu_combine:
                x1_ref[...] = jnp.maximum(p_ref[0] + p_ref[1] + b_ref[...],
                                          0.0)
            else:
                x1_ref[...] = p_ref[0]

        out_ref[0] = lax.dot_general(
            x1_ref[...], w_ref[0], (((1,), (0,)), ((), ())),
            precision=lax.Precision.HIGHEST,
            preferred_element_type=jnp.float32)

    return pl.pallas_call(
        body,
        grid=(nrel,),
        in_specs=[
            pl.BlockSpec(P.shape, lambda i: (0, 0, 0)),
            pl.BlockSpec((1,) + W.shape[1:], lambda i: (i, 0, 0)),
            pl.BlockSpec(bsum.shape, lambda i: (0, 0)),
        ],
        out_specs=pl.BlockSpec((1, nrows, W.shape[2]), lambda i: (i, 0, 0)),
        out_shape=jax.ShapeDtypeStruct((nrel, nrows, W.shape[2]),
                                       jnp.float32),
        scratch_shapes=[pltpu.VMEM((nrows, P.shape[2]), jnp.float32)],
    )(P, W, bsum)


def _heads(Q, b2s, A_w, A_b, C_w, C_b2):
    RB = 1000
    nblk = _N // RB

    def body(q_ref, b_ref, aw_ref, ab_ref, cw_ref, cb_ref,
             o0, o1, o2, o3, o4, oc):
        emb = jnp.maximum(q_ref[0] + q_ref[1] + b_ref[...], 0.0)
        outs = (o0, o1, o2, o3, o4)
        for r in range(_NREL):
            outs[r][...] = lax.dot_general(
                emb, aw_ref[r], (((1,), (1,)), ((), ())),
                precision=lax.Precision.HIGHEST,
                preferred_element_type=jnp.float32) + ab_ref[r]
        logits = lax.dot_general(
            emb, cw_ref[...], (((1,), (1,)), ((), ())),
            precision=lax.Precision.HIGHEST,
            preferred_element_type=jnp.float32) + cb_ref[...]
        m = jnp.max(logits, axis=1, keepdims=True)
        lse = m + jnp.log(jnp.sum(jnp.exp(logits - m), axis=1,
                                  keepdims=True))
        oc[...] = logits - lse

    link_shape = jax.ShapeDtypeStruct((_N, _NHID), jnp.float32)
    return pl.pallas_call(
        body,
        grid=(nblk,),
        in_specs=[
            pl.BlockSpec((2, RB, _NHID), lambda i: (0, i, 0)),
            pl.BlockSpec((1, _NHID), lambda i: (0, 0)),
            pl.BlockSpec(A_w.shape, lambda i: (0, 0, 0)),
            pl.BlockSpec(A_b.shape, lambda i: (0, 0)),
            pl.BlockSpec(C_w.shape, lambda i: (0, 0)),
            pl.BlockSpec((1, _NCLASS), lambda i: (0, 0)),
        ],
        out_specs=[
            pl.BlockSpec((RB, _NHID), lambda i: (i, 0)),
            pl.BlockSpec((RB, _NHID), lambda i: (i, 0)),
            pl.BlockSpec((RB, _NHID), lambda i: (i, 0)),
            pl.BlockSpec((RB, _NHID), lambda i: (i, 0)),
            pl.BlockSpec((RB, _NHID), lambda i: (i, 0)),
            pl.BlockSpec((RB, _NCLASS), lambda i: (i, 0)),
        ],
        out_shape=[link_shape, link_shape, link_shape, link_shape,
                   link_shape,
                   jax.ShapeDtypeStruct((_N, _NCLASS), jnp.float32)],
    )(Q, b2s, A_w, A_b, C_w, C_b2)


def kernel(x, adjs_edge_index, adjs_edge_weight, W1, b1, W2, b2,
           A_w, A_b, C_w, C_b):
    offs = (jnp.arange(_NADJ, dtype=jnp.int32) * _N)[:, None]
    src2d = (adjs_edge_index[:, 1, :] + offs).reshape(_NADJ * _CPR, _CHUNK)
    dst2d = adjs_edge_index[:, 0, :].reshape(_NADJ * _CPR, _CHUNK)
    w2d = adjs_edge_weight.reshape(_NADJ * _CPR, _CHUNK)
    pad_i = jnp.zeros((_PADC, _CHUNK), jnp.int32)
    src2d = jnp.concatenate([src2d, pad_i], axis=0)
    dst2d = jnp.concatenate([dst2d, pad_i], axis=0)
    w2d = jnp.concatenate([w2d, jnp.zeros((_PADC, _CHUNK), jnp.float32)],
                          axis=0)
    b1s = jnp.sum(b1, axis=0, keepdims=True)
    b2s = jnp.sum(b2, axis=0, keepdims=True)
    zb = jnp.zeros((1, _NHID), jnp.float32)

    S1 = _mm_rel(x.reshape(1, _N, _NFEAT), W1, zb, relu_combine=False)
    P1 = _sc_agg(S1.reshape(_NADJ * _N, _NHID), src2d, dst2d, w2d)
    S2 = _mm_rel(P1.reshape(2, _N, _NHID), W2, b1s, relu_combine=True)
    P2 = _sc_agg(S2.reshape(_NADJ * _N, _NHID), src2d, dst2d, w2d)
    o0, o1, o2, o3, o4, oc = _heads(P2.reshape(2, _N, _NHID), b2s,
                                    A_w, A_b, C_w,
                                    C_b.reshape(1, _NCLASS))
    return (o0, o1, o2, o3, o4, oc)


# speedup vs baseline: 6.3058x; 1.5527x over previous
import dataclasses
import functools

import jax
import jax.numpy as jnp
from jax import lax
from jax.experimental import pallas as pl
from jax.experimental.pallas import tpu as pltpu
from jax.experimental.pallas import tpu_sc as plsc

_N = 10000
_NFEAT = 128
_NHID = 128
_NCLASS = 2
_NREL = 5
_NADJ = 11
_E = 320000

_LANES = 16
_CHUNK = 128
_CPR = _E // _CHUNK
_CPP = 32
_TOTC = 27520
_PADC = _TOTC - _NADJ * _CPR
_UNITS = _TOTC // _CPP
_NW = 32
_UPW = (_UNITS + _NW - 1) // _NW
_SUB_ROWS = 624
_TAIL_ROWS = _N - 16 * _SUB_ROWS
_NSEG = _NHID // _LANES


def _sc_body(S_hbm, src_hbm, dst_hbm, w_hbm, out_hbm, src_v, dst_v, w_v,
             rows_v, rows_w, acc, gsem0, gsem1):
    cid = lax.axis_index("c")
    sid = lax.axis_index("s")
    wid = sid * 2 + cid
    zeros16 = jnp.zeros((_LANES,), jnp.float32)

    @pl.loop(0, _CHUNK)
    def _(rr):
        for j in range(_NSEG):
            rows_v[rr, pl.ds(j * _LANES, _LANES)] = zeros16

    base = sid * _SUB_ROWS
    for z in range(_SUB_ROWS // _CHUNK):
        pltpu.sync_copy(rows_v, acc.at[pl.ds(base + z * _CHUNK, _CHUNK)])
    rem = _SUB_ROWS % _CHUNK
    if rem:
        pltpu.sync_copy(rows_v.at[pl.ds(0, rem)],
                        acc.at[pl.ds(base + _SUB_ROWS - rem, rem)])

    @pl.when(sid == 0)
    def _():
        pltpu.sync_copy(rows_v.at[pl.ds(0, _TAIL_ROWS)],
                        acc.at[pl.ds(16 * _SUB_ROWS, _TAIL_ROWS)])

    plsc.subcore_barrier()

    @pl.loop(0, _UPW)
    def _(t):
        u = wid + t * _NW

        @pl.when(u < _UNITS)
        def _():
            c0 = u * _CPP
            pltpu.sync_copy(src_hbm.at[pl.ds(c0, _CPP)], src_v)
            pltpu.sync_copy(dst_hbm.at[pl.ds(c0, _CPP)], dst_v)
            pltpu.sync_copy(w_hbm.at[pl.ds(c0, _CPP)], w_v)

            pltpu.async_copy(S_hbm.at[src_v.at[0]], rows_v, gsem0)
            pltpu.async_copy(S_hbm.at[src_v.at[1]], rows_w, gsem1)

            @pl.loop(0, _CPP, step=2)
            def _(c):
                for b, buf, sem in ((0, rows_v, gsem0), (1, rows_w, gsem1)):
                    cc = c + b
                    pltpu.make_async_copy(
                        S_hbm.at[src_v.at[cc]], buf, sem).wait()
                    cidx = jnp.full((_LANES,), cc, jnp.int32)

                    @pl.loop(0, _CHUNK)
                    def _(e):
                        wv = plsc.load_gather(
                            w_v, [cidx, jnp.full((_LANES,), e, jnp.int32)])
                        for j in range(_NSEG):
                            sl = pl.ds(j * _LANES, _LANES)
                            buf[e, sl] = buf[e, sl] * wv

                    pltpu.sync_copy(buf, acc.at[dst_v.at[cc]], add=True)

                    @pl.when(cc + 2 < _CPP)
                    def _():
                        pltpu.async_copy(
                            S_hbm.at[src_v.at[cc + 2]], buf, sem)

    plsc.subcore_barrier()
    pltpu.sync_copy(acc.at[pl.ds(sid * _SUB_ROWS, _SUB_ROWS)],
                    out_hbm.at[pl.ds(cid * _N + sid * _SUB_ROWS, _SUB_ROWS)])

    @pl.when(sid == 0)
    def _():
        pltpu.sync_copy(acc.at[pl.ds(16 * _SUB_ROWS, _TAIL_ROWS)],
                        out_hbm.at[pl.ds(cid * _N + 16 * _SUB_ROWS,
                                         _TAIL_ROWS)])


def _sc_agg(S2d, src2d, dst2d, w2d):
    mesh = plsc.VectorSubcoreMesh(core_axis_name="c", subcore_axis_name="s")
    cp = pltpu.CompilerParams()
    if "needs_layout_passes" in pltpu.CompilerParams.__dataclass_fields__:
        cp = dataclasses.replace(cp, needs_layout_passes=False)
    kern = functools.partial(
        pl.kernel,
        compiler_params=cp,
        out_type=jax.ShapeDtypeStruct((2 * _N, _NHID), jnp.float32),
        mesh=mesh,
        scratch_types=[
            pltpu.VMEM((_CPP, _CHUNK), jnp.int32),
            pltpu.VMEM((_CPP, _CHUNK), jnp.int32),
            pltpu.VMEM((_CPP, _CHUNK), jnp.float32),
            pltpu.VMEM((_CHUNK, _NHID), jnp.float32),
            pltpu.VMEM((_CHUNK, _NHID), jnp.float32),
            pltpu.VMEM_SHARED((_N, _NHID), jnp.float32),
            pltpu.SemaphoreType.DMA,
            pltpu.SemaphoreType.DMA,
        ],
    )(_sc_body)
    return kern(S2d, src2d, dst2d, w2d)


def _mm_rel(P, W, bsum, relu_combine):
    nrel = W.shape[0]
    nrows = P.shape[1]

    def body(p_ref, w_ref, b_ref, out_ref, x1_ref):
        @pl.when(pl.program_id(0) == 0)
        def _():
            if relu_combine:
                x1_ref[...] = jnp.maximum(p_ref[0] + p_ref[1] + b_ref[...],
                                          0.0)
            else:
                x1_ref[...] = p_ref[0]

        out_ref[0] = lax.dot_general(
            x1_ref[...], w_ref[0], (((1,), (0,)), ((), ())),
            precision=lax.Precision.HIGHEST,
            preferred_element_type=jnp.float32)

    return pl.pallas_call(
        body,
        grid=(nrel,),
        in_specs=[
            pl.BlockSpec(P.shape, lambda i: (0, 0, 0)),
            pl.BlockSpec((1,) + W.shape[1:], lambda i: (i, 0, 0)),
            pl.BlockSpec(bsum.shape, lambda i: (0, 0)),
        ],
        out_specs=pl.BlockSpec((1, nrows, W.shape[2]), lambda i: (i, 0, 0)),
        out_shape=jax.ShapeDtypeStruct((nrel, nrows, W.shape[2]),
                                       jnp.float32),
        scratch_shapes=[pltpu.VMEM((nrows, P.shape[2]), jnp.float32)],
    )(P, W, bsum)


def _heads(Q, b2s, A_w, A_b, C_w, C_b2):
    RB = 1000
    nblk = _N // RB

    def body(q_ref, b_ref, aw_ref, ab_ref, cw_ref, cb_ref,
             o0, o1, o2, o3, o4, oc):
        emb = jnp.maximum(q_ref[0] + q_ref[1] + b_ref[...], 0.0)
        outs = (o0, o1, o2, o3, o4)
        for r in range(_NREL):
            outs[r][...] = lax.dot_general(
                emb, aw_ref[r], (((1,), (1,)), ((), ())),
                precision=lax.Precision.HIGHEST,
                preferred_element_type=jnp.float32) + ab_ref[r]
        logits = lax.dot_general(
            emb, cw_ref[...], (((1,), (1,)), ((), ())),
            precision=lax.Precision.HIGHEST,
            preferred_element_type=jnp.float32) + cb_ref[...]
        m = jnp.max(logits, axis=1, keepdims=True)
        lse = m + jnp.log(jnp.sum(jnp.exp(logits - m), axis=1,
                                  keepdims=True))
        oc[...] = logits - lse

    link_shape = jax.ShapeDtypeStruct((_N, _NHID), jnp.float32)
    return pl.pallas_call(
        body,
        grid=(nblk,),
        in_specs=[
            pl.BlockSpec((2, RB, _NHID), lambda i: (0, i, 0)),
            pl.BlockSpec((1, _NHID), lambda i: (0, 0)),
            pl.BlockSpec(A_w.shape, lambda i: (0, 0, 0)),
            pl.BlockSpec(A_b.shape, lambda i: (0, 0)),
            pl.BlockSpec(C_w.shape, lambda i: (0, 0)),
            pl.BlockSpec((1, _NCLASS), lambda i: (0, 0)),
        ],
        out_specs=[
            pl.BlockSpec((RB, _NHID), lambda i: (i, 0)),
            pl.BlockSpec((RB, _NHID), lambda i: (i, 0)),
            pl.BlockSpec((RB, _NHID), lambda i: (i, 0)),
            pl.BlockSpec((RB, _NHID), lambda i: (i, 0)),
            pl.BlockSpec((RB, _NHID), lambda i: (i, 0)),
            pl.BlockSpec((RB, _NCLASS), lambda i: (i, 0)),
        ],
        out_shape=[link_shape, link_shape, link_shape, link_shape,
                   link_shape,
                   jax.ShapeDtypeStruct((_N, _NCLASS), jnp.float32)],
    )(Q, b2s, A_w, A_b, C_w, C_b2)


def kernel(x, adjs_edge_index, adjs_edge_weight, W1, b1, W2, b2,
           A_w, A_b, C_w, C_b):
    offs = (jnp.arange(_NADJ, dtype=jnp.int32) * _N)[:, None]
    src2d = (adjs_edge_index[:, 1, :] + offs).reshape(_NADJ * _CPR, _CHUNK)
    dst2d = adjs_edge_index[:, 0, :].reshape(_NADJ * _CPR, _CHUNK)
    w2d = adjs_edge_weight.reshape(_NADJ * _CPR, _CHUNK)
    pad_i = jnp.zeros((_PADC, _CHUNK), jnp.int32)
    src2d = jnp.concatenate([src2d, pad_i], axis=0)
    dst2d = jnp.concatenate([dst2d, pad_i], axis=0)
    w2d = jnp.concatenate([w2d, jnp.zeros((_PADC, _CHUNK), jnp.float32)],
                          axis=0)
    b1s = jnp.sum(b1, axis=0, keepdims=True)
    b2s = jnp.sum(b2, axis=0, keepdims=True)
    zb = jnp.zeros((1, _NHID), jnp.float32)

    S1 = _mm_rel(x.reshape(1, _N, _NFEAT), W1, zb, relu_combine=False)
    P1 = _sc_agg(S1.reshape(_NADJ * _N, _NHID), src2d, dst2d, w2d)
    S2 = _mm_rel(P1.reshape(2, _N, _NHID), W2, b1s, relu_combine=True)
    P2 = _sc_agg(S2.reshape(_NADJ * _N, _NHID), src2d, dst2d, w2d)
    o0, o1, o2, o3, o4, oc = _heads(P2.reshape(2, _N, _NHID), b2s,
                                    A_w, A_b, C_w,
                                    C_b.reshape(1, _NCLASS))
    return (o0, o1, o2, o3, o4, oc)


# speedup vs baseline: 7.9563x; 1.2617x over previous
import dataclasses
import functools

import jax
import jax.numpy as jnp
from jax import lax
from jax.experimental import pallas as pl
from jax.experimental.pallas import tpu as pltpu
from jax.experimental.pallas import tpu_sc as plsc

_N = 10000
_NFEAT = 128
_NHID = 128
_NCLASS = 2
_NREL = 5
_NADJ = 11
_E = 320000

_LANES = 16
_CHUNK = 128
_CPR = _E // _CHUNK
_CPP = 32
_TOTC = 27520
_PADC = _TOTC - _NADJ * _CPR
_UNITS = _TOTC // _CPP
_NW = 32
_UPW = (_UNITS + _NW - 1) // _NW
_SUB_ROWS = 624
_TAIL_ROWS = _N - 16 * _SUB_ROWS
_NSEG = _NHID // _LANES


_HALF = 64


def _sc_body(S_hbm, src_hbm, dst_hbm, w_hbm, out_hbm, src_v, dst_v, w_v,
             rows_v, rows_w, acc, gsem0, gsem1, ssem0, ssem1):
    cid = lax.axis_index("c")
    sid = lax.axis_index("s")
    wid = sid * 2 + cid
    zeros16 = jnp.zeros((_LANES,), jnp.float32)

    @pl.loop(0, _CHUNK)
    def _(rr):
        for j in range(_NSEG):
            rows_v[rr, pl.ds(j * _LANES, _LANES)] = zeros16

    base = sid * _SUB_ROWS
    for z in range(_SUB_ROWS // _CHUNK):
        pltpu.sync_copy(rows_v, acc.at[pl.ds(base + z * _CHUNK, _CHUNK)])
    rem = _SUB_ROWS % _CHUNK
    if rem:
        pltpu.sync_copy(rows_v.at[pl.ds(0, rem)],
                        acc.at[pl.ds(base + _SUB_ROWS - rem, rem)])

    @pl.when(sid == 0)
    def _():
        pltpu.sync_copy(rows_v.at[pl.ds(0, _TAIL_ROWS)],
                        acc.at[pl.ds(16 * _SUB_ROWS, _TAIL_ROWS)])

    plsc.subcore_barrier()

    @pl.loop(0, _UPW)
    def _(t):
        u = wid + t * _NW

        @pl.when(u < _UNITS)
        def _():
            c0 = u * _CPP
            pltpu.sync_copy(src_hbm.at[pl.ds(c0, _CPP)], src_v)
            pltpu.sync_copy(dst_hbm.at[pl.ds(2 * c0, 2 * _CPP)], dst_v)
            pltpu.sync_copy(w_hbm.at[pl.ds(c0, _CPP)], w_v)

            pltpu.async_copy(S_hbm.at[src_v.at[0]], rows_v, gsem0)
            pltpu.async_copy(S_hbm.at[src_v.at[1]], rows_w, gsem1)

            @pl.loop(0, _CPP, step=2)
            def _(c):
                for b, buf, gsem, ssem in ((0, rows_v, gsem0, ssem0),
                                           (1, rows_w, gsem1, ssem1)):
                    cc = c + b
                    pltpu.make_async_copy(
                        S_hbm.at[src_v.at[cc]], buf, gsem).wait()
                    cidx = jnp.full((_LANES,), cc, jnp.int32)
                    for h in range(_CHUNK // _HALF):

                        @pl.loop(h * _HALF, (h + 1) * _HALF, step=2)
                        def _(e):
                            for u in range(2):
                                wv = plsc.load_gather(
                                    w_v, [cidx,
                                          jnp.full((_LANES,), e + u,
                                                   jnp.int32)])
                                for j in range(_NSEG):
                                    sl = pl.ds(j * _LANES, _LANES)
                                    buf[e + u, sl] = buf[e + u, sl] * wv

                        pltpu.async_copy(
                            buf.at[pl.ds(h * _HALF, _HALF)],
                            acc.at[dst_v.at[2 * cc + h]], ssem, add=True)

                    @pl.when(cc >= 2)
                    def _():
                        for h in range(_CHUNK // _HALF):
                            pltpu.make_async_copy(
                                buf.at[pl.ds(h * _HALF, _HALF)],
                                acc.at[dst_v.at[0]], ssem).wait()

                    @pl.when(cc + 2 < _CPP)
                    def _():
                        pltpu.async_copy(
                            S_hbm.at[src_v.at[cc + 2]], buf, gsem)

            for b, buf, ssem in ((0, rows_v, ssem0), (1, rows_w, ssem1)):
                for h in range(_CHUNK // _HALF):
                    pltpu.make_async_copy(
                        buf.at[pl.ds(h * _HALF, _HALF)],
                        acc.at[dst_v.at[0]], ssem).wait()

    plsc.subcore_barrier()
    pltpu.sync_copy(acc.at[pl.ds(sid * _SUB_ROWS, _SUB_ROWS)],
                    out_hbm.at[pl.ds(cid * _N + sid * _SUB_ROWS, _SUB_ROWS)])

    @pl.when(sid == 0)
    def _():
        pltpu.sync_copy(acc.at[pl.ds(16 * _SUB_ROWS, _TAIL_ROWS)],
                        out_hbm.at[pl.ds(cid * _N + 16 * _SUB_ROWS,
                                         _TAIL_ROWS)])


def _sc_agg(S2d, src2d, dst2d, w2d):
    mesh = plsc.VectorSubcoreMesh(core_axis_name="c", subcore_axis_name="s")
    cp = pltpu.CompilerParams()
    if "needs_layout_passes" in pltpu.CompilerParams.__dataclass_fields__:
        cp = dataclasses.replace(cp, needs_layout_passes=False)
    kern = functools.partial(
        pl.kernel,
        compiler_params=cp,
        out_type=jax.ShapeDtypeStruct((2 * _N, _NHID), jnp.float32),
        mesh=mesh,
        scratch_types=[
            pltpu.VMEM((_CPP, _CHUNK), jnp.int32),
            pltpu.VMEM((2 * _CPP, _HALF), jnp.int32),
            pltpu.VMEM((_CPP, _CHUNK), jnp.float32),
            pltpu.VMEM((_CHUNK, _NHID), jnp.float32),
            pltpu.VMEM((_CHUNK, _NHID), jnp.float32),
            pltpu.VMEM_SHARED((_N, _NHID), jnp.float32),
            pltpu.SemaphoreType.DMA,
            pltpu.SemaphoreType.DMA,
            pltpu.SemaphoreType.DMA,
            pltpu.SemaphoreType.DMA,
        ],
    )(_sc_body)
    return kern(S2d, src2d, dst2d, w2d)


def _mm_rel(P, W, bsum, relu_combine):
    nrel = W.shape[0]
    nrows = P.shape[1]

    def body(p_ref, w_ref, b_ref, out_ref, x1_ref):
        @pl.when(pl.program_id(0) == 0)
        def _():
            if relu_combine:
                x1_ref[...] = jnp.maximum(p_ref[0] + p_ref[1] + b_ref[...],
                                          0.0)
            else:
                x1_ref[...] = p_ref[0]

        out_ref[0] = lax.dot_general(
            x1_ref[...], w_ref[0], (((1,), (0,)), ((), ())),
            precision=lax.Precision.HIGHEST,
            preferred_element_type=jnp.float32)

    return pl.pallas_call(
        body,
        grid=(nrel,),
        in_specs=[
            pl.BlockSpec(P.shape, lambda i: (0, 0, 0)),
            pl.BlockSpec((1,) + W.shape[1:], lambda i: (i, 0, 0)),
            pl.BlockSpec(bsum.shape, lambda i: (0, 0)),
        ],
        out_specs=pl.BlockSpec((1, nrows, W.shape[2]), lambda i: (i, 0, 0)),
        out_shape=jax.ShapeDtypeStruct((nrel, nrows, W.shape[2]),
                                       jnp.float32),
        scratch_shapes=[pltpu.VMEM((nrows, P.shape[2]), jnp.float32)],
    )(P, W, bsum)


def _heads(Q, b2s, A_w, A_b, C_w, C_b2):
    RB = 1000
    nblk = _N // RB

    def body(q_ref, b_ref, aw_ref, ab_ref, cw_ref, cb_ref,
             o0, o1, o2, o3, o4, oc):
        emb = jnp.maximum(q_ref[0] + q_ref[1] + b_ref[...], 0.0)
        outs = (o0, o1, o2, o3, o4)
        for r in range(_NREL):
            outs[r][...] = lax.dot_general(
                emb, aw_ref[r], (((1,), (1,)), ((), ())),
                precision=lax.Precision.HIGHEST,
                preferred_element_type=jnp.float32) + ab_ref[r]
        logits = lax.dot_general(
            emb, cw_ref[...], (((1,), (1,)), ((), ())),
            precision=lax.Precision.HIGHEST,
            preferred_element_type=jnp.float32) + cb_ref[...]
        m = jnp.max(logits, axis=1, keepdims=True)
        lse = m + jnp.log(jnp.sum(jnp.exp(logits - m), axis=1,
                                  keepdims=True))
        oc[...] = logits - lse

    link_shape = jax.ShapeDtypeStruct((_N, _NHID), jnp.float32)
    return pl.pallas_call(
        body,
        grid=(nblk,),
        in_specs=[
            pl.BlockSpec((2, RB, _NHID), lambda i: (0, i, 0)),
            pl.BlockSpec((1, _NHID), lambda i: (0, 0)),
            pl.BlockSpec(A_w.shape, lambda i: (0, 0, 0)),
            pl.BlockSpec(A_b.shape, lambda i: (0, 0)),
            pl.BlockSpec(C_w.shape, lambda i: (0, 0)),
            pl.BlockSpec((1, _NCLASS), lambda i: (0, 0)),
        ],
        out_specs=[
            pl.BlockSpec((RB, _NHID), lambda i: (i, 0)),
            pl.BlockSpec((RB, _NHID), lambda i: (i, 0)),
            pl.BlockSpec((RB, _NHID), lambda i: (i, 0)),
            pl.BlockSpec((RB, _NHID), lambda i: (i, 0)),
            pl.BlockSpec((RB, _NHID), lambda i: (i, 0)),
            pl.BlockSpec((RB, _NCLASS), lambda i: (i, 0)),
        ],
        out_shape=[link_shape, link_shape, link_shape, link_shape,
                   link_shape,
                   jax.ShapeDtypeStruct((_N, _NCLASS), jnp.float32)],
    )(Q, b2s, A_w, A_b, C_w, C_b2)


def kernel(x, adjs_edge_index, adjs_edge_weight, W1, b1, W2, b2,
           A_w, A_b, C_w, C_b):
    offs = (jnp.arange(_NADJ, dtype=jnp.int32) * _N)[:, None]
    src2d = (adjs_edge_index[:, 1, :] + offs).reshape(_NADJ * _CPR, _CHUNK)
    dst2d = adjs_edge_index[:, 0, :].reshape(2 * _NADJ * _CPR, _HALF)
    w2d = adjs_edge_weight.reshape(_NADJ * _CPR, _CHUNK)
    pad_i = jnp.zeros((_PADC, _CHUNK), jnp.int32)
    src2d = jnp.concatenate([src2d, pad_i], axis=0)
    dst2d = jnp.concatenate(
        [dst2d, jnp.zeros((2 * _PADC, _HALF), jnp.int32)], axis=0)
    w2d = jnp.concatenate([w2d, jnp.zeros((_PADC, _CHUNK), jnp.float32)],
                          axis=0)
    b1s = jnp.sum(b1, axis=0, keepdims=True)
    b2s = jnp.sum(b2, axis=0, keepdims=True)
    zb = jnp.zeros((1, _NHID), jnp.float32)

    S1 = _mm_rel(x.reshape(1, _N, _NFEAT), W1, zb, relu_combine=False)
    P1 = _sc_agg(S1.reshape(_NADJ * _N, _NHID), src2d, dst2d, w2d)
    S2 = _mm_rel(P1.reshape(2, _N, _NHID), W2, b1s, relu_combine=True)
    P2 = _sc_agg(S2.reshape(_NADJ * _N, _NHID), src2d, dst2d, w2d)
    o0, o1, o2, o3, o4, oc = _heads(P2.reshape(2, _N, _NHID), b2s,
                                    A_w, A_b, C_w,
                                    C_b.reshape(1, _NCLASS))
    return (o0, o1, o2, o3, o4, oc)


# speedup vs baseline: 8.7062x; 1.0942x over previous
import dataclasses
import functools

import jax
import jax.numpy as jnp
from jax import lax
from jax.experimental import pallas as pl
from jax.experimental.pallas import tpu as pltpu
from jax.experimental.pallas import tpu_sc as plsc

_N = 10000
_NFEAT = 128
_NHID = 128
_NCLASS = 2
_NREL = 5
_NADJ = 11
_E = 320000

_LANES = 16
_CHUNK = 128
_CPR = _E // _CHUNK
_CPP = 32
_TOTC = 27520
_PADC = _TOTC - _NADJ * _CPR
_UNITS = _TOTC // _CPP
_NW = 32
_UPW = (_UNITS + _NW - 1) // _NW
_SUB_ROWS = 624
_TAIL_ROWS = _N - 16 * _SUB_ROWS
_NSEG = _NHID // _LANES


_HALF = 64


def _sc_body(S_hbm, src_hbm, dst_hbm, w_hbm, out_hbm, src_v, dst_v, w_v,
             rows_v, rows_w, acc, gsem0, gsem1, ssem0, ssem1):
    cid = lax.axis_index("c")
    sid = lax.axis_index("s")
    wid = sid * 2 + cid
    zeros16 = jnp.zeros((_LANES,), jnp.float32)

    @pl.loop(0, _CHUNK)
    def _(rr):
        for j in range(_NSEG):
            rows_v[rr, pl.ds(j * _LANES, _LANES)] = zeros16

    base = sid * _SUB_ROWS
    for z in range(_SUB_ROWS // _CHUNK):
        pltpu.sync_copy(rows_v, acc.at[pl.ds(base + z * _CHUNK, _CHUNK)])
    rem = _SUB_ROWS % _CHUNK
    if rem:
        pltpu.sync_copy(rows_v.at[pl.ds(0, rem)],
                        acc.at[pl.ds(base + _SUB_ROWS - rem, rem)])

    @pl.when(sid == 0)
    def _():
        pltpu.sync_copy(rows_v.at[pl.ds(0, _TAIL_ROWS)],
                        acc.at[pl.ds(16 * _SUB_ROWS, _TAIL_ROWS)])

    plsc.subcore_barrier()

    @pl.loop(0, _UPW)
    def _(t):
        u = wid + t * _NW

        @pl.when(u < _UNITS)
        def _():
            c0 = u * _CPP
            pltpu.sync_copy(src_hbm.at[pl.ds(c0, _CPP)], src_v)
            pltpu.sync_copy(dst_hbm.at[pl.ds(2 * c0, 2 * _CPP)], dst_v)
            pltpu.sync_copy(w_hbm.at[pl.ds(c0, _CPP)], w_v)

            pltpu.async_copy(S_hbm.at[src_v.at[0]], rows_v, gsem0)
            pltpu.async_copy(S_hbm.at[src_v.at[1]], rows_w, gsem1)

            @pl.loop(0, _CPP, step=2)
            def _(c):
                for b, buf, gsem, ssem in ((0, rows_v, gsem0, ssem0),
                                           (1, rows_w, gsem1, ssem1)):
                    cc = c + b
                    pltpu.make_async_copy(
                        S_hbm.at[src_v.at[cc]], buf, gsem).wait()
                    for h in range(_CHUNK // _HALF):

                        @pl.loop(h * _HALF, (h + 1) * _HALF, step=_LANES)
                        def _(e):
                            wseg = w_v[cc, pl.ds(e, _LANES)]
                            for u in range(_LANES):
                                wv = lax.gather(
                                    wseg,
                                    jnp.full((_LANES, 1), u, jnp.int32),
                                    lax.GatherDimensionNumbers(
                                        offset_dims=(),
                                        collapsed_slice_dims=(0,),
                                        start_index_map=(0,)),
                                    (1,),
                                    mode=lax.GatherScatterMode
                                    .PROMISE_IN_BOUNDS)
                                for j in range(_NSEG):
                                    sl = pl.ds(j * _LANES, _LANES)
                                    buf[e + u, sl] = buf[e + u, sl] * wv

                        pltpu.async_copy(
                            buf.at[pl.ds(h * _HALF, _HALF)],
                            acc.at[dst_v.at[2 * cc + h]], ssem, add=True)

                    @pl.when(cc >= 2)
                    def _():
                        for h in range(_CHUNK // _HALF):
                            pltpu.make_async_copy(
                                buf.at[pl.ds(h * _HALF, _HALF)],
                                acc.at[dst_v.at[0]], ssem).wait()

                    @pl.when(cc + 2 < _CPP)
                    def _():
                        pltpu.async_copy(
                            S_hbm.at[src_v.at[cc + 2]], buf, gsem)

            for b, buf, ssem in ((0, rows_v, ssem0), (1, rows_w, ssem1)):
                for h in range(_CHUNK // _HALF):
                    pltpu.make_async_copy(
                        buf.at[pl.ds(h * _HALF, _HALF)],
                        acc.at[dst_v.at[0]], ssem).wait()

    plsc.subcore_barrier()
    pltpu.sync_copy(acc.at[pl.ds(sid * _SUB_ROWS, _SUB_ROWS)],
                    out_hbm.at[pl.ds(cid * _N + sid * _SUB_ROWS, _SUB_ROWS)])

    @pl.when(sid == 0)
    def _():
        pltpu.sync_copy(acc.at[pl.ds(16 * _SUB_ROWS, _TAIL_ROWS)],
                        out_hbm.at[pl.ds(cid * _N + 16 * _SUB_ROWS,
                                         _TAIL_ROWS)])


def _sc_agg(S2d, src2d, dst2d, w2d):
    mesh = plsc.VectorSubcoreMesh(core_axis_name="c", subcore_axis_name="s")
    cp = pltpu.CompilerParams()
    if "needs_layout_passes" in pltpu.CompilerParams.__dataclass_fields__:
        cp = dataclasses.replace(cp, needs_layout_passes=False)
    kern = functools.partial(
        pl.kernel,
        compiler_params=cp,
        out_type=jax.ShapeDtypeStruct((2 * _N, _NHID), jnp.float32),
        mesh=mesh,
        scratch_types=[
            pltpu.VMEM((_CPP, _CHUNK), jnp.int32),
            pltpu.VMEM((2 * _CPP, _HALF), jnp.int32),
            pltpu.VMEM((_CPP, _CHUNK), jnp.float32),
            pltpu.VMEM((_CHUNK, _NHID), jnp.float32),
            pltpu.VMEM((_CHUNK, _NHID), jnp.float32),
            pltpu.VMEM_SHARED((_N, _NHID), jnp.float32),
            pltpu.SemaphoreType.DMA,
            pltpu.SemaphoreType.DMA,
            pltpu.SemaphoreType.DMA,
            pltpu.SemaphoreType.DMA,
        ],
    )(_sc_body)
    return kern(S2d, src2d, dst2d, w2d)


def _mm_rel(P, W, bsum, relu_combine):
    nrel = W.shape[0]
    nrows = P.shape[1]

    def body(p_ref, w_ref, b_ref, out_ref, x1_ref):
        @pl.when(pl.program_id(0) == 0)
        def _():
            if relu_combine:
                x1_ref[...] = jnp.maximum(p_ref[0] + p_ref[1] + b_ref[...],
                                          0.0)
            else:
                x1_ref[...] = p_ref[0]

        out_ref[0] = lax.dot_general(
            x1_ref[...], w_ref[0], (((1,), (0,)), ((), ())),
            precision=lax.Precision.HIGHEST,
            preferred_element_type=jnp.float32)

    return pl.pallas_call(
        body,
        grid=(nrel,),
        in_specs=[
            pl.BlockSpec(P.shape, lambda i: (0, 0, 0)),
            pl.BlockSpec((1,) + W.shape[1:], lambda i: (i, 0, 0)),
            pl.BlockSpec(bsum.shape, lambda i: (0, 0)),
        ],
        out_specs=pl.BlockSpec((1, nrows, W.shape[2]), lambda i: (i, 0, 0)),
        out_shape=jax.ShapeDtypeStruct((nrel, nrows, W.shape[2]),
                                       jnp.float32),
        scratch_shapes=[pltpu.VMEM((nrows, P.shape[2]), jnp.float32)],
    )(P, W, bsum)


def _heads(Q, b2s, A_w, A_b, C_w, C_b2):
    RB = 1000
    nblk = _N // RB

    def body(q_ref, b_ref, aw_ref, ab_ref, cw_ref, cb_ref,
             o0, o1, o2, o3, o4, oc):
        emb = jnp.maximum(q_ref[0] + q_ref[1] + b_ref[...], 0.0)
        outs = (o0, o1, o2, o3, o4)
        for r in range(_NREL):
            outs[r][...] = lax.dot_general(
                emb, aw_ref[r], (((1,), (1,)), ((), ())),
                precision=lax.Precision.HIGHEST,
                preferred_element_type=jnp.float32) + ab_ref[r]
        logits = lax.dot_general(
            emb, cw_ref[...], (((1,), (1,)), ((), ())),
            precision=lax.Precision.HIGHEST,
            preferred_element_type=jnp.float32) + cb_ref[...]
        m = jnp.max(logits, axis=1, keepdims=True)
        lse = m + jnp.log(jnp.sum(jnp.exp(logits - m), axis=1,
                                  keepdims=True))
        oc[...] = logits - lse

    link_shape = jax.ShapeDtypeStruct((_N, _NHID), jnp.float32)
    return pl.pallas_call(
        body,
        grid=(nblk,),
        in_specs=[
            pl.BlockSpec((2, RB, _NHID), lambda i: (0, i, 0)),
            pl.BlockSpec((1, _NHID), lambda i: (0, 0)),
            pl.BlockSpec(A_w.shape, lambda i: (0, 0, 0)),
            pl.BlockSpec(A_b.shape, lambda i: (0, 0)),
            pl.BlockSpec(C_w.shape, lambda i: (0, 0)),
            pl.BlockSpec((1, _NCLASS), lambda i: (0, 0)),
        ],
        out_specs=[
            pl.BlockSpec((RB, _NHID), lambda i: (i, 0)),
            pl.BlockSpec((RB, _NHID), lambda i: (i, 0)),
            pl.BlockSpec((RB, _NHID), lambda i: (i, 0)),
            pl.BlockSpec((RB, _NHID), lambda i: (i, 0)),
            pl.BlockSpec((RB, _NHID), lambda i: (i, 0)),
            pl.BlockSpec((RB, _NCLASS), lambda i: (i, 0)),
        ],
        out_shape=[link_shape, link_shape, link_shape, link_shape,
                   link_shape,
                   jax.ShapeDtypeStruct((_N, _NCLASS), jnp.float32)],
    )(Q, b2s, A_w, A_b, C_w, C_b2)


def kernel(x, adjs_edge_index, adjs_edge_weight, W1, b1, W2, b2,
           A_w, A_b, C_w, C_b):
    offs = (jnp.arange(_NADJ, dtype=jnp.int32) * _N)[:, None]
    src2d = (adjs_edge_index[:, 1, :] + offs).reshape(_NADJ * _CPR, _CHUNK)
    dst2d = adjs_edge_index[:, 0, :].reshape(2 * _NADJ * _CPR, _HALF)
    w2d = adjs_edge_weight.reshape(_NADJ * _CPR, _CHUNK)
    pad_i = jnp.zeros((_PADC, _CHUNK), jnp.int32)
    src2d = jnp.concatenate([src2d, pad_i], axis=0)
    dst2d = jnp.concatenate(
        [dst2d, jnp.zeros((2 * _PADC, _HALF), jnp.int32)], axis=0)
    w2d = jnp.concatenate([w2d, jnp.zeros((_PADC, _CHUNK), jnp.float32)],
                          axis=0)
    b1s = jnp.sum(b1, axis=0, keepdims=True)
    b2s = jnp.sum(b2, axis=0, keepdims=True)
    zb = jnp.zeros((1, _NHID), jnp.float32)

    S1 = _mm_rel(x.reshape(1, _N, _NFEAT), W1, zb, relu_combine=False)
    P1 = _sc_agg(S1.reshape(_NADJ * _N, _NHID), src2d, dst2d, w2d)
    S2 = _mm_rel(P1.reshape(2, _N, _NHID), W2, b1s, relu_combine=True)
    P2 = _sc_agg(S2.reshape(_NADJ * _N, _NHID), src2d, dst2d, w2d)
    o0, o1, o2, o3, o4, oc = _heads(P2.reshape(2, _N, _NHID), b2s,
                                    A_w, A_b, C_w,
                                    C_b.reshape(1, _NCLASS))
    return (o0, o1, o2, o3, o4, oc)
